# baseline (speedup 1.0000x reference)
import functools
import math

import jax
import jax.numpy as jnp
from jax import lax
from jax.experimental import pallas as pl
from jax.experimental.pallas import tpu as pltpu
from jax.experimental.pallas import tpu_sc as plsc

_DIM = 64
_PADDIM = 128
_SCALE = math.sqrt(_DIM)
_LANES = 16

_NC = 2
_NS = 16
_NW = _NC * _NS

_GBUF = 8
_LOOK = 6
_OBUF = 4


def _make_lookup(n_sent, seq):
    assert n_sent % _NW == 0
    sent_w = n_sent // _NW
    idx_w = sent_w * seq
    mesh = plsc.VectorSubcoreMesh(core_axis_name="c", subcore_axis_name="s")
    vecs = (seq + _LANES - 1) // _LANES

    @functools.partial(
        pl.kernel,
        mesh=mesh,
        out_type=jax.ShapeDtypeStruct((n_sent, seq, _DIM), jnp.float32),
        scratch_types=[
            pltpu.VMEM((idx_w + _LANES,), jnp.int32),
            pltpu.VMEM((_GBUF, vecs * _LANES, _PADDIM), jnp.float32),
            pltpu.VMEM((_OBUF, seq, _DIM), jnp.float32),
            pltpu.SemaphoreType.DMA((_GBUF,)),
            pltpu.SemaphoreType.DMA((_OBUF,)),
        ],
        compiler_params=pltpu.CompilerParams(use_tc_tiling_on_sc=True),
    )
    def lookup(idx_hbm, table_hbm, out_hbm, idx_v, rows_v, stage_v, gsem, osem):
        wid = lax.axis_index("s") * _NC + lax.axis_index("c")
        wbase = wid * idx_w

        pltpu.sync_copy(idx_hbm.at[pl.ds(wbase, idx_w)], idx_v.at[pl.ds(0, idx_w)])
        idx_v[pl.ds(idx_w, _LANES)] = jnp.zeros((_LANES,), jnp.int32)

        def gather_sent(t, start):
            s = lax.rem(t, _GBUF)
            copies = []
            for j in range(vecs):
                iv = idx_v[pl.ds(t * seq + j * _LANES, _LANES)]
                c = pltpu.make_async_copy(
                    table_hbm.at[iv],
                    rows_v.at[s, pl.ds(j * _LANES, _LANES)],
                    gsem.at[s],
                )
                if start:
                    c.start()
                copies.append(c)
            return copies

        def out_copy(t):
            ss = lax.rem(t, _OBUF)
            return pltpu.make_async_copy(
                stage_v.at[ss],
                out_hbm.at[wid * sent_w + t],
                osem.at[ss],
            )

        for t in range(_LOOK):
            gather_sent(t, True)

        def sent_body(t, carry):
            s = lax.rem(t, _GBUF)
            ss = lax.rem(t, _OBUF)

            @pl.when(t + _LOOK < sent_w)
            def _fire_ahead():
                gather_sent(t + _LOOK, True)

            for c in gather_sent(t, False):
                c.wait()

            @pl.when(t >= _OBUF)
            def _drain_out():
                out_copy(t - _OBUF).wait()

            @plsc.parallel_loop(0, seq, step=1, unroll=8)
            def _scale(r):
                for cc in range(_DIM // _LANES):
                    sl = pl.ds(cc * _LANES, _LANES)
                    stage_v[ss, r, sl] = rows_v[s, r, sl] * _SCALE

            out_copy(t).start()
            return carry

        lax.fori_loop(0, sent_w, sent_body, 0)

        for t in range(sent_w - _OBUF, sent_w):
            out_copy(t).wait()

    return lookup


def kernel(x, table):
    n_sent, seq = x.shape
    idx_flat = x.reshape(n_sent * seq).astype(jnp.int32)
    table_pad = jnp.pad(table, ((0, 0), (0, _PADDIM - _DIM)))
    out = _make_lookup(n_sent, seq)(idx_flat, table_pad)
    return lax.optimization_barrier(out)


# speedup vs baseline: 1.0017x; 1.0017x over previous
import functools
import math

import jax
import jax.numpy as jnp
from jax import lax
from jax.experimental import pallas as pl
from jax.experimental.pallas import tpu as pltpu
from jax.experimental.pallas import tpu_sc as plsc

_DIM = 64
_PADDIM = 128
_SCALE = math.sqrt(_DIM)
_LANES = 16

_NC = 2
_NS = 16
_NW = _NC * _NS

_GBUF = 10
_LOOK = 8
_OBUF = 2


def _make_lookup(n_sent, seq):
    assert n_sent % _NW == 0
    sent_w = n_sent // _NW
    idx_w = sent_w * seq
    mesh = plsc.VectorSubcoreMesh(core_axis_name="c", subcore_axis_name="s")
    vecs = (seq + _LANES - 1) // _LANES

    @functools.partial(
        pl.kernel,
        mesh=mesh,
        out_type=jax.ShapeDtypeStruct((n_sent, seq, _DIM), jnp.float32),
        scratch_types=[
            pltpu.VMEM((idx_w + _LANES,), jnp.int32),
            pltpu.VMEM((_GBUF, vecs * _LANES, _PADDIM), jnp.float32),
            pltpu.VMEM((_OBUF, seq, _DIM), jnp.float32),
            pltpu.SemaphoreType.DMA((_GBUF,)),
            pltpu.SemaphoreType.DMA((_OBUF,)),
        ],
        compiler_params=pltpu.CompilerParams(use_tc_tiling_on_sc=True),
    )
    def lookup(idx_hbm, table_hbm, out_hbm, idx_v, rows_v, stage_v, gsem, osem):
        wid = lax.axis_index("s") * _NC + lax.axis_index("c")
        wbase = wid * idx_w

        pltpu.sync_copy(idx_hbm.at[pl.ds(wbase, idx_w)], idx_v.at[pl.ds(0, idx_w)])
        idx_v[pl.ds(idx_w, _LANES)] = jnp.zeros((_LANES,), jnp.int32)

        def gather_sent(t, start):
            s = lax.rem(t, _GBUF)
            copies = []
            for j in range(vecs):
                iv = idx_v[pl.ds(t * seq + j * _LANES, _LANES)]
                c = pltpu.make_async_copy(
                    table_hbm.at[iv],
                    rows_v.at[s, pl.ds(j * _LANES, _LANES)],
                    gsem.at[s],
                )
                if start:
                    c.start()
                copies.append(c)
            return copies

        def out_copy(t):
            ss = lax.rem(t, _OBUF)
            return pltpu.make_async_copy(
                stage_v.at[ss],
                out_hbm.at[wid * sent_w + t],
                osem.at[ss],
            )

        for t in range(_LOOK):
            gather_sent(t, True)

        def sent_body(t, carry):
            s = lax.rem(t, _GBUF)
            ss = lax.rem(t, _OBUF)

            @pl.when(t + _LOOK < sent_w)
            def _fire_ahead():
                gather_sent(t + _LOOK, True)

            for c in gather_sent(t, False):
                c.wait()

            @pl.when(t >= _OBUF)
            def _drain_out():
                out_copy(t - _OBUF).wait()

            @plsc.parallel_loop(0, seq, step=1, unroll=8)
            def _scale(r):
                for cc in range(_DIM // _LANES):
                    sl = pl.ds(cc * _LANES, _LANES)
                    stage_v[ss, r, sl] = rows_v[s, r, sl] * _SCALE

            out_copy(t).start()
            return carry

        lax.fori_loop(0, sent_w, sent_body, 0)

        for t in range(sent_w - _OBUF, sent_w):
            out_copy(t).wait()

    return lookup


def kernel(x, table):
    n_sent, seq = x.shape
    idx_flat = x.reshape(n_sent * seq).astype(jnp.int32)
    table_pad = jnp.pad(table, ((0, 0), (0, _PADDIM - _DIM)))
    out = _make_lookup(n_sent, seq)(idx_flat, table_pad)
    return lax.optimization_barrier(out)
